# baseline (speedup 1.0000x reference)
import jax
import jax.numpy as jnp
from jax import lax
from jax.experimental import pallas as pl
from jax.experimental.pallas import tpu as pltpu
from jax.experimental.pallas import tpu_sc as plsc

NUM_ENVS = 256
MAX_STEPS = 64
ZONE_SLOTS = 128
GAME_INFO_DIM = 256
MAX_OPTIONS = 32
OPT_SCALAR_DIM = 16
B = 256

R_SO = 0
R_ST = R_SO + 16384
R_GI = R_ST + 16384
R_OS = R_GI + 32768
R_OM = R_OS + 65536
R_MAY = R_OM + 4096
R_OLP = R_MAY + 128
R_VAL = R_OLP + 128
R_LH = R_VAL + 128
R_LC = R_LH + 32768
R_TOTAL = R_LC + 32768

i32 = jnp.int32
f32 = jnp.float32


def _pipe(ac, items, bufs, sis, sos):
  nb = len(bufs)
  n = len(items)
  din = [None] * n
  dout = [None] * n

  def fire_in(j):
    src, srow, _, _, nr = items[j]
    return ac(src.at[pl.ds(srow, nr)], bufs[j % nb].at[pl.ds(0, nr)],
              sis[j % nb])

  def fire_out(i):
    _, _, dref, drow, nr = items[i]
    return ac(bufs[i % nb].at[pl.ds(0, nr)], dref.at[pl.ds(drow, nr)],
              sos[i % nb])

  for k in range(min(nb - 1, n)):
    din[k] = fire_in(k)
  for i in range(n):
    if din[i] is None:
      din[i] = fire_in(i)
    din[i].wait()
    dout[i] = fire_out(i)
    j = i + nb - 1
    if j < n and din[j] is None:
      if i >= 1:
        dout[i - 1].wait()
      din[j] = fire_in(j)
  for i in range(max(0, n - nb + 1), n):
    dout[i].wait()


def _body(step_pad,
          mso, bso, mst, bst, mgi, bgi, mos, bos, mom, bom,
          mlh, blh, mlc, blc,
          mcr, bcr, mok, bok,
          mmay, bmay, molp, bolp, mval, bval,
          mtr, btr, mpe, bpe, mpp, bpp, msc,
          fs, cr_out, tr_out, pe_out, ok_out, pp_out, sc_out,
          s_vm, sost_sb, gi_sb, os_sb, lh_sb, lc_sb, cr_sb,
          om_old, ok_old, omv, okv,
          mayb, olpb, valb, mayv, olpv, valv,
          trb, peb, ppb, trv, pev, ppv, scb,
          i_sost, i_gi, i_os, i_lh, i_lc, i_cr, i_omg, i_omsc, i_okg,
          fb0, fb1, fb2, fb3, ib0, ib1,
          sem_sm, sem_g,
          sfi0, sfi1, sfi2, sfi3, sfo0, sfo1, sfo2, sfo3,
          sii0, sii1, sio0, sio1, sem_fin):
  w = lax.axis_index("s") * 2 + lax.axis_index("c")
  ac = pltpu.async_copy

  sm = [ac(step_pad.at[pl.ds(8 * w, 16)], s_vm, sem_sm),
        ac(bso.at[pl.ds(8 * w, 8)], sost_sb.at[pl.ds(0, 8)], sem_sm),
        ac(bst.at[pl.ds(8 * w, 8)], sost_sb.at[pl.ds(8, 8)], sem_sm),
        ac(bgi.at[pl.ds(16 * w, 16)], gi_sb, sem_sm),
        ac(bos.at[pl.ds(32 * w, 32)], os_sb, sem_sm),
        ac(blh.at[pl.ds(16 * w, 16)], lh_sb, sem_sm),
        ac(blc.at[pl.ds(16 * w, 16)], lc_sb, sem_sm),
        ac(bcr.at[pl.ds(8 * w, 8)], cr_sb.at[pl.ds(0, 8)], sem_sm),
        ac(bcr.at[pl.ds(8 * w, 8)], cr_sb.at[pl.ds(8, 8)], sem_sm),
        ac(bom.at[pl.ds(2 * w, 2)], omv, sem_sm),
        ac(bok.at[pl.ds(2 * w, 2)], okv, sem_sm)]
  trio_f = [(bmay, mayb, mayv, R_MAY, mmay),
            (bolp, olpb, olpv, R_OLP, molp),
            (bval, valb, valv, R_VAL, mval)]
  trio_i = [(btr, trb, trv, tr_out, mtr),
            (bpe, peb, pev, pe_out, mpe),
            (bpp, ppb, ppv, pp_out, mpp)]
  for bref, buf, vbuf, _, mref in trio_f + trio_i:
    sm.append(ac(mref.at[pl.ds(4 * w, 4)], buf, sem_sm))
    sm.append(ac(bref.at[pl.ds(8 * w, 16)], vbuf, sem_sm))

  @pl.when(w < 16)
  def _():
    pltpu.sync_copy(msc.at[pl.ds(16 * w, 16)], scb)
    scb[...] = scb[...] + 1
    pltpu.sync_copy(scb, sc_out.at[pl.ds(16 * w, 16)])

  for d in sm:
    d.wait()

  lane = lax.iota(i32, 16)
  s_vec = s_vm[...]
  e7 = lane & 7

  def bc(v, pat):
    return v.at[pat].get(mode="promise_in_bounds")

  s8 = bc(s_vec, e7)

  g_idx = (8 * w + e7) * 16 + (s8 >> 2)
  i_omg[...] = g_idx
  i_omsc[...] = R_OM + g_idx
  i_okg[...] = g_idx
  gth = [ac(mom.at[i_omg], om_old, sem_g), ac(mok.at[i_okg], ok_old, sem_g)]

  i_sost[...] = jnp.where(lane < 8, R_SO, R_ST) + (8 * w + e7) * 64 + s8
  el2 = lane >> 1
  s2 = bc(s_vec, el2)
  gi_rows = (8 * w + el2) * 128 + s2 * 2 + (lane & 1)
  i_gi[...] = R_GI + gi_rows
  i_lh[...] = R_LH + gi_rows
  i_lc[...] = R_LC + gi_rows
  for k in range(2):
    ll = lane + 16 * k
    el4 = ll >> 2
    s4 = bc(s_vec, el4)
    i_os[pl.ds(16 * k, 16)] = R_OS + (8 * w + el4) * 256 + s4 * 4 + (ll & 3)
  i_cr[...] = (8 * w + e7) * 64 + s8

  f_items = []
  for src, base_row, rpw in ((mso, R_SO, 512), (mst, R_ST, 512),
                             (mgi, R_GI, 1024), (mos, R_OS, 2048),
                             (mom, R_OM, 128), (mlh, R_LH, 1024),
                             (mlc, R_LC, 1024)):
    for c in range(rpw // 128):
      f_items.append((src, w * rpw + 128 * c, fs, base_row + w * rpw + 128 * c,
                      128))
  _pipe(ac, f_items, [fb0, fb1, fb2, fb3], [sfi0, sfi1, sfi2, sfi3],
        [sfo0, sfo1, sfo2, sfo3])

  i_items = []
  for src, dref, rpw in ((mcr, cr_out, 512), (mok, ok_out, 128)):
    for c in range(rpw // 128):
      i_items.append((src, w * rpw + 128 * c, dref, w * rpw + 128 * c, 128))
  _pipe(ac, i_items, [ib0, ib1], [sii0, sii1], [sio0, sio1])

  for d in gth:
    d.wait()
  for old, vals in ((om_old, omv), (ok_old, okv)):
    for r in range(16):
      le = r & 7
      t = bc(s_vec, jnp.full((16,), le, i32)) & 3
      vrow = le >> 2
      c0 = (le & 3) * 32
      for k in range(8):
        pos = jnp.full((16,), 16 * k, i32) + lane
        cond = (pos >> 5) == t
        val = vals[vrow, pl.ds(c0 + 16 * (k & 1), 16)]
        old[r, pl.ds(16 * k, 16)] = jnp.where(cond, val,
                                              old[r, pl.ds(16 * k, 16)])

  for _, buf, vbuf, _, _ in trio_f + trio_i:
    vals = vbuf[...]
    for r in range(4):
      s_e = bc(s_vec, jnp.full((16,), 2 * r, i32))
      v_e = bc(vals, jnp.full((16,), 2 * r, i32))
      s_o = bc(s_vec, jnp.full((16,), 2 * r + 1, i32))
      v_o = bc(vals, jnp.full((16,), 2 * r + 1, i32))
      for k in range(8):
        pos = jnp.full((16,), 16 * k, i32) + lane
        old_v = buf[r, pl.ds(16 * k, 16)]
        buf[r, pl.ds(16 * k, 16)] = jnp.where(
            pos == s_e, v_e, jnp.where(pos == 64 + s_o, v_o, old_v))

  fin = []
  for _, buf, _, base, _ in trio_f:
    fin.append(ac(buf, fs.at[pl.ds(base + 4 * w, 4)], sem_fin))
  for _, buf, _, outref, _ in trio_i:
    fin.append(ac(buf, outref.at[pl.ds(4 * w, 4)], sem_fin))
  fin.append(ac(sost_sb, fs.at[i_sost], sem_fin))
  fin.append(ac(gi_sb, fs.at[i_gi], sem_fin))
  fin.append(ac(os_sb, fs.at[i_os], sem_fin))
  fin.append(ac(lh_sb, fs.at[i_lh], sem_fin))
  fin.append(ac(lc_sb, fs.at[i_lc], sem_fin))
  fin.append(ac(cr_sb, cr_out.at[i_cr], sem_fin))
  fin.append(ac(om_old, fs.at[i_omsc], sem_fin))
  fin.append(ac(ok_old, ok_out.at[i_okg], sem_fin))
  for d in fin:
    d.wait()


def kernel(env_indices, step_indices, slot_card_rows, slot_occupied,
           slot_tapped, game_info, trace_kind_id, pending_kind_id,
           option_kind_ids, option_scalars, option_mask, may_selected,
           old_log_probs, values, perspective_player_idx, lstm_h_in,
           lstm_c_in, mem_slot_card_rows, mem_slot_occupied, mem_slot_tapped,
           mem_game_info, mem_trace_kind_id, mem_pending_kind_id,
           mem_option_kind_ids, mem_option_scalars, mem_option_mask,
           mem_may_selected, mem_old_log_prob, mem_value,
           mem_perspective_player_idx, mem_lstm_h, mem_lstm_c, mem_step_count):
  del env_indices
  pad16 = lambda a: jnp.pad(a, (0, 16))

  def env_major_rows(x, chi):
    u = x.reshape(chi, 128, 256)
    return jnp.transpose(u, (2, 0, 1)).reshape(256 * chi, 128)

  mos_arg = jax.lax.reshape(
      jax.lax.transpose(mem_option_scalars, (1, 2, 3, 0)),
      (65536, 128), dimensions=(3, 0, 1, 2))
  mom_arg = env_major_rows(
      jax.lax.transpose(mem_option_mask, (1, 2, 0)), 16)
  mok_arg = env_major_rows(
      jax.lax.transpose(mem_option_kind_ids, (1, 2, 0)), 16)
  bos_arg = env_major_rows(
      jax.lax.transpose(option_scalars, (1, 2, 0)), 4)

  def env_major_sub(x, k):
    h = 128 // k
    u = jax.lax.transpose(x, (1, 0)).reshape(k, 256 // h, h)
    return jnp.transpose(u, (1, 2, 0)).reshape(256 * k // 128, 128)

  bom_arg = env_major_sub(option_mask, 32)
  bok_arg = env_major_sub(option_kind_ids, 32)

  def env_major_64(x):
    u = jax.lax.transpose(x, (1, 0)).reshape(64, 128, 2)
    return jnp.transpose(u, (1, 2, 0)).reshape(128, 128)

  mmay_arg = env_major_64(mem_may_selected)
  molp_arg = env_major_64(mem_old_log_prob)
  mval_arg = env_major_64(mem_value)
  mtr_arg = env_major_64(mem_trace_kind_id)
  mpe_arg = env_major_64(mem_pending_kind_id)
  mpp_arg = env_major_64(mem_perspective_player_idx)

  out_type = (jax.ShapeDtypeStruct((R_TOTAL, 128), f32),
              jax.ShapeDtypeStruct((16384, 128), i32),
              jax.ShapeDtypeStruct((128, 128), i32),
              jax.ShapeDtypeStruct((128, 128), i32),
              jax.ShapeDtypeStruct((4096, 128), i32),
              jax.ShapeDtypeStruct((128, 128), i32),
              jax.ShapeDtypeStruct((NUM_ENVS,), i32))
  scratch = [pltpu.VMEM((16,), i32),
             pltpu.VMEM((16, 128), f32), pltpu.VMEM((16, 128), f32),
             pltpu.VMEM((32, 128), f32), pltpu.VMEM((16, 128), f32),
             pltpu.VMEM((16, 128), f32), pltpu.VMEM((16, 128), i32),
             pltpu.VMEM((16, 128), f32), pltpu.VMEM((16, 128), i32),
             pltpu.VMEM((2, 128), f32), pltpu.VMEM((2, 128), i32),
             pltpu.VMEM((4, 128), f32), pltpu.VMEM((4, 128), f32),
             pltpu.VMEM((4, 128), f32),
             pltpu.VMEM((16,), f32), pltpu.VMEM((16,), f32),
             pltpu.VMEM((16,), f32),
             pltpu.VMEM((4, 128), i32), pltpu.VMEM((4, 128), i32),
             pltpu.VMEM((4, 128), i32),
             pltpu.VMEM((16,), i32), pltpu.VMEM((16,), i32),
             pltpu.VMEM((16,), i32),
             pltpu.VMEM((16,), i32),
             pltpu.VMEM((16,), i32), pltpu.VMEM((16,), i32),
             pltpu.VMEM((32,), i32), pltpu.VMEM((16,), i32),
             pltpu.VMEM((16,), i32), pltpu.VMEM((16,), i32),
             pltpu.VMEM((16,), i32), pltpu.VMEM((16,), i32),
             pltpu.VMEM((16,), i32),
             pltpu.VMEM((128, 128), f32), pltpu.VMEM((128, 128), f32),
             pltpu.VMEM((128, 128), f32), pltpu.VMEM((128, 128), f32),
             pltpu.VMEM((128, 128), i32), pltpu.VMEM((128, 128), i32),
             pltpu.SemaphoreType.DMA, pltpu.SemaphoreType.DMA,
             pltpu.SemaphoreType.DMA, pltpu.SemaphoreType.DMA,
             pltpu.SemaphoreType.DMA, pltpu.SemaphoreType.DMA,
             pltpu.SemaphoreType.DMA, pltpu.SemaphoreType.DMA,
             pltpu.SemaphoreType.DMA, pltpu.SemaphoreType.DMA,
             pltpu.SemaphoreType.DMA, pltpu.SemaphoreType.DMA,
             pltpu.SemaphoreType.DMA, pltpu.SemaphoreType.DMA,
             pltpu.SemaphoreType.DMA]
  run = pl.kernel(
      _body, out_type=out_type,
      mesh=plsc.VectorSubcoreMesh(core_axis_name="c", subcore_axis_name="s"),
      scratch_types=scratch)

  fs, cr, tr, pe, ok, pp, sc = run(
      pad16(step_indices),
      mem_slot_occupied.reshape(16384, 128), slot_occupied,
      mem_slot_tapped.reshape(16384, 128), slot_tapped,
      mem_game_info.reshape(32768, 128), game_info.reshape(512, 128),
      mos_arg, bos_arg,
      mom_arg, bom_arg,
      mem_lstm_h.reshape(32768, 128), lstm_h_in.reshape(512, 128),
      mem_lstm_c.reshape(32768, 128), lstm_c_in.reshape(512, 128),
      mem_slot_card_rows.reshape(16384, 128), slot_card_rows,
      mok_arg, bok_arg,
      mmay_arg, pad16(may_selected),
      molp_arg, pad16(old_log_probs),
      mval_arg, pad16(values),
      mtr_arg, pad16(trace_kind_id),
      mpe_arg, pad16(pending_kind_id),
      mpp_arg, pad16(perspective_player_idx),
      mem_step_count)
  return (fs.reshape(-1),
          cr.reshape(NUM_ENVS, MAX_STEPS, ZONE_SLOTS),
          tr.reshape(NUM_ENVS, MAX_STEPS),
          pe.reshape(NUM_ENVS, MAX_STEPS),
          ok.reshape(NUM_ENVS, MAX_STEPS, MAX_OPTIONS),
          pp.reshape(NUM_ENVS, MAX_STEPS),
          sc)


# speedup vs baseline: 2.0186x; 2.0186x over previous
import jax
import jax.numpy as jnp
from jax import lax
from jax.experimental import pallas as pl
from jax.experimental.pallas import tpu as pltpu
from jax.experimental.pallas import tpu_sc as plsc

NUM_ENVS = 256
MAX_STEPS = 64
ZONE_SLOTS = 128
GAME_INFO_DIM = 256
MAX_OPTIONS = 32
OPT_SCALAR_DIM = 16
B = 256

R_SO = 0
R_ST = R_SO + 16384
R_GI = R_ST + 16384
R_OS = R_GI + 32768
R_OM = R_OS + 65536
R_MAY = R_OM + 4096
R_OLP = R_MAY + 128
R_VAL = R_OLP + 128
R_LH = R_VAL + 128
R_LC = R_LH + 32768
R_TOTAL = R_LC + 32768

i32 = jnp.int32
f32 = jnp.float32


def _pipe(ac, items, bufs, sis, sos):
  nb = len(bufs)
  n = len(items)
  din = [None] * n
  dout = [None] * n

  def fire_in(j):
    src, srow, _, _, nr = items[j]
    return ac(src.at[pl.ds(srow, nr)], bufs[j % nb].at[pl.ds(0, nr)],
              sis[j % nb])

  def fire_out(i):
    _, _, dref, drow, nr = items[i]
    return ac(bufs[i % nb].at[pl.ds(0, nr)], dref.at[pl.ds(drow, nr)],
              sos[i % nb])

  for k in range(min(nb - 1, n)):
    din[k] = fire_in(k)
  for i in range(n):
    if din[i] is None:
      din[i] = fire_in(i)
    din[i].wait()
    dout[i] = fire_out(i)
    j = i + nb - 1
    if j < n and din[j] is None:
      if i >= 1:
        dout[i - 1].wait()
      din[j] = fire_in(j)
  for i in range(max(0, n - nb + 1), n):
    dout[i].wait()


def _body(step_pad,
          mso, bso, mst, bst, mgi, bgi, mos, bos, mom, bom,
          mlh, blh, mlc, blc,
          mcr, bcr, mok, bok,
          mmay, bmay, molp, bolp, mval, bval,
          mtr, btr, mpe, bpe, mpp, bpp, msc,
          fs, cr_out, tr_out, pe_out, ok_out, pp_out, sc_out,
          s_vm, sost_sb, gi_sb, os_sb, lh_sb, lc_sb, cr_sb,
          om_old, ok_old, omv, okv,
          mayb, olpb, valb, mayv, olpv, valv,
          trb, peb, ppb, trv, pev, ppv, scb,
          i_sost, i_gi, i_os, i_lh, i_lc, i_cr, i_omg, i_omsc, i_okg,
          fb0, fb1, fb2, fb3, ib0, ib1,
          sem_sm, sem_g,
          sfi0, sfi1, sfi2, sfi3, sfo0, sfo1, sfo2, sfo3,
          sii0, sii1, sio0, sio1, sem_fin):
  w = lax.axis_index("s") * 2 + lax.axis_index("c")
  ac = pltpu.async_copy

  sm = [ac(step_pad.at[pl.ds(8 * w, 16)], s_vm, sem_sm),
        ac(bso.at[pl.ds(8 * w, 8)], sost_sb.at[pl.ds(0, 8)], sem_sm),
        ac(bst.at[pl.ds(8 * w, 8)], sost_sb.at[pl.ds(8, 8)], sem_sm),
        ac(bgi.at[pl.ds(16 * w, 16)], gi_sb, sem_sm),
        ac(bos.at[pl.ds(32 * w, 32)], os_sb, sem_sm),
        ac(blh.at[pl.ds(16 * w, 16)], lh_sb, sem_sm),
        ac(blc.at[pl.ds(16 * w, 16)], lc_sb, sem_sm),
        ac(bcr.at[pl.ds(8 * w, 8)], cr_sb.at[pl.ds(0, 8)], sem_sm),
        ac(bcr.at[pl.ds(8 * w, 8)], cr_sb.at[pl.ds(8, 8)], sem_sm),
        ac(bom.at[pl.ds(2 * w, 2)], omv, sem_sm),
        ac(bok.at[pl.ds(2 * w, 2)], okv, sem_sm)]
  trio_f = [(bmay, mayb, mayv, R_MAY, mmay),
            (bolp, olpb, olpv, R_OLP, molp),
            (bval, valb, valv, R_VAL, mval)]
  trio_i = [(btr, trb, trv, tr_out, mtr),
            (bpe, peb, pev, pe_out, mpe),
            (bpp, ppb, ppv, pp_out, mpp)]
  for bref, buf, vbuf, _, mref in trio_f + trio_i:
    sm.append(ac(mref.at[pl.ds(4 * w, 4)], buf, sem_sm))
    sm.append(ac(bref.at[pl.ds(8 * w, 16)], vbuf, sem_sm))

  @pl.when(w < 16)
  def _():
    pltpu.sync_copy(msc.at[pl.ds(16 * w, 16)], scb)
    scb[...] = scb[...] + 1
    pltpu.sync_copy(scb, sc_out.at[pl.ds(16 * w, 16)])

  for d in sm:
    d.wait()

  lane = lax.iota(i32, 16)
  s_vec = s_vm[...]
  e7 = lane & 7

  def bc(v, pat):
    return v.at[pat].get(mode="promise_in_bounds")

  s8 = bc(s_vec, e7)

  g_idx = (8 * w + e7) * 16 + (s8 >> 2)
  i_omg[...] = g_idx
  i_omsc[...] = R_OM + g_idx
  i_okg[...] = g_idx
  gth = [ac(mom.at[i_omg], om_old, sem_g), ac(mok.at[i_okg], ok_old, sem_g)]

  i_sost[...] = jnp.where(lane < 8, R_SO, R_ST) + (8 * w + e7) * 64 + s8
  el2 = lane >> 1
  s2 = bc(s_vec, el2)
  gi_rows = (8 * w + el2) * 128 + s2 * 2 + (lane & 1)
  i_gi[...] = R_GI + gi_rows
  i_lh[...] = R_LH + gi_rows
  i_lc[...] = R_LC + gi_rows
  for k in range(2):
    ll = lane + 16 * k
    el4 = ll >> 2
    s4 = bc(s_vec, el4)
    i_os[pl.ds(16 * k, 16)] = R_OS + (8 * w + el4) * 256 + s4 * 4 + (ll & 3)
  i_cr[...] = (8 * w + e7) * 64 + s8

  f_items = []
  for src, base_row, rpw in ((mso, R_SO, 512), (mst, R_ST, 512),
                             (mgi, R_GI, 1024), (mos, R_OS, 2048),
                             (mom, R_OM, 128), (mlh, R_LH, 1024),
                             (mlc, R_LC, 1024)):
    for c in range(rpw // 128):
      f_items.append((src, w * rpw + 128 * c, fs, base_row + w * rpw + 128 * c,
                      128))
  _pipe(ac, f_items, [fb0, fb1, fb2, fb3], [sfi0, sfi1, sfi2, sfi3],
        [sfo0, sfo1, sfo2, sfo3])

  for d in gth:
    d.wait()
  for old, vals in ((om_old, omv), (ok_old, okv)):
    for r in range(16):
      le = r & 7
      t = bc(s_vec, jnp.full((16,), le, i32)) & 3
      vrow = le >> 2
      c0 = (le & 3) * 32
      for k in range(8):
        pos = jnp.full((16,), 16 * k, i32) + lane
        cond = (pos >> 5) == t
        val = vals[vrow, pl.ds(c0 + 16 * (k & 1), 16)]
        old[r, pl.ds(16 * k, 16)] = jnp.where(cond, val,
                                              old[r, pl.ds(16 * k, 16)])

  for _, buf, vbuf, _, _ in trio_f + trio_i:
    vals = vbuf[...]
    for r in range(4):
      s_e = bc(s_vec, jnp.full((16,), 2 * r, i32))
      v_e = bc(vals, jnp.full((16,), 2 * r, i32))
      s_o = bc(s_vec, jnp.full((16,), 2 * r + 1, i32))
      v_o = bc(vals, jnp.full((16,), 2 * r + 1, i32))
      for k in range(8):
        pos = jnp.full((16,), 16 * k, i32) + lane
        old_v = buf[r, pl.ds(16 * k, 16)]
        buf[r, pl.ds(16 * k, 16)] = jnp.where(
            pos == s_e, v_e, jnp.where(pos == 64 + s_o, v_o, old_v))

  fin = []
  for _, buf, _, base, _ in trio_f:
    fin.append(ac(buf, fs.at[pl.ds(base + 4 * w, 4)], sem_fin))
  for _, buf, _, outref, _ in trio_i:
    fin.append(ac(buf, outref.at[pl.ds(4 * w, 4)], sem_fin))
  fin.append(ac(sost_sb, fs.at[i_sost], sem_fin))
  fin.append(ac(gi_sb, fs.at[i_gi], sem_fin))
  fin.append(ac(os_sb, fs.at[i_os], sem_fin))
  fin.append(ac(lh_sb, fs.at[i_lh], sem_fin))
  fin.append(ac(lc_sb, fs.at[i_lc], sem_fin))
  fin.append(ac(om_old, fs.at[i_omsc], sem_fin))

  i_items = []
  for src, dref, rpw in ((mcr, cr_out, 512), (mok, ok_out, 128)):
    for c in range(rpw // 128):
      i_items.append((src, w * rpw + 128 * c, dref, w * rpw + 128 * c, 128))
  _pipe(ac, i_items, [ib0, ib1], [sii0, sii1], [sio0, sio1])

  fin.append(ac(cr_sb, cr_out.at[i_cr], sem_fin))
  fin.append(ac(ok_old, ok_out.at[i_okg], sem_fin))
  for d in fin:
    d.wait()


def kernel(env_indices, step_indices, slot_card_rows, slot_occupied,
           slot_tapped, game_info, trace_kind_id, pending_kind_id,
           option_kind_ids, option_scalars, option_mask, may_selected,
           old_log_probs, values, perspective_player_idx, lstm_h_in,
           lstm_c_in, mem_slot_card_rows, mem_slot_occupied, mem_slot_tapped,
           mem_game_info, mem_trace_kind_id, mem_pending_kind_id,
           mem_option_kind_ids, mem_option_scalars, mem_option_mask,
           mem_may_selected, mem_old_log_prob, mem_value,
           mem_perspective_player_idx, mem_lstm_h, mem_lstm_c, mem_step_count):
  del env_indices
  pad16 = lambda a: jnp.pad(a, (0, 16))

  def env_major_rows(x, chi):
    u = x.reshape(chi, 128, 256)
    return jnp.transpose(u, (2, 0, 1)).reshape(256 * chi, 128)

  mos_arg = env_major_rows(
      jax.lax.transpose(mem_option_scalars, (1, 2, 3, 0)), 256)
  mom_arg = env_major_rows(
      jax.lax.transpose(mem_option_mask, (1, 2, 0)), 16)
  mok_arg = env_major_rows(
      jax.lax.transpose(mem_option_kind_ids, (1, 2, 0)), 16)
  bos_arg = env_major_rows(
      jax.lax.transpose(option_scalars, (1, 2, 0)), 4)

  def env_major_sub(x, k):
    h = 128 // k
    u = jax.lax.transpose(x, (1, 0)).reshape(k, 256 // h, h)
    return jnp.transpose(u, (1, 2, 0)).reshape(256 * k // 128, 128)

  bom_arg = env_major_sub(option_mask, 32)
  bok_arg = env_major_sub(option_kind_ids, 32)

  def env_major_64(x):
    u = jax.lax.transpose(x, (1, 0)).reshape(64, 128, 2)
    return jnp.transpose(u, (1, 2, 0)).reshape(128, 128)

  mmay_arg = env_major_64(mem_may_selected)
  molp_arg = env_major_64(mem_old_log_prob)
  mval_arg = env_major_64(mem_value)
  mtr_arg = env_major_64(mem_trace_kind_id)
  mpe_arg = env_major_64(mem_pending_kind_id)
  mpp_arg = env_major_64(mem_perspective_player_idx)

  out_type = (jax.ShapeDtypeStruct((R_TOTAL, 128), f32),
              jax.ShapeDtypeStruct((16384, 128), i32),
              jax.ShapeDtypeStruct((128, 128), i32),
              jax.ShapeDtypeStruct((128, 128), i32),
              jax.ShapeDtypeStruct((4096, 128), i32),
              jax.ShapeDtypeStruct((128, 128), i32),
              jax.ShapeDtypeStruct((NUM_ENVS,), i32))
  scratch = [pltpu.VMEM((16,), i32),
             pltpu.VMEM((16, 128), f32), pltpu.VMEM((16, 128), f32),
             pltpu.VMEM((32, 128), f32), pltpu.VMEM((16, 128), f32),
             pltpu.VMEM((16, 128), f32), pltpu.VMEM((16, 128), i32),
             pltpu.VMEM((16, 128), f32), pltpu.VMEM((16, 128), i32),
             pltpu.VMEM((2, 128), f32), pltpu.VMEM((2, 128), i32),
             pltpu.VMEM((4, 128), f32), pltpu.VMEM((4, 128), f32),
             pltpu.VMEM((4, 128), f32),
             pltpu.VMEM((16,), f32), pltpu.VMEM((16,), f32),
             pltpu.VMEM((16,), f32),
             pltpu.VMEM((4, 128), i32), pltpu.VMEM((4, 128), i32),
             pltpu.VMEM((4, 128), i32),
             pltpu.VMEM((16,), i32), pltpu.VMEM((16,), i32),
             pltpu.VMEM((16,), i32),
             pltpu.VMEM((16,), i32),
             pltpu.VMEM((16,), i32), pltpu.VMEM((16,), i32),
             pltpu.VMEM((32,), i32), pltpu.VMEM((16,), i32),
             pltpu.VMEM((16,), i32), pltpu.VMEM((16,), i32),
             pltpu.VMEM((16,), i32), pltpu.VMEM((16,), i32),
             pltpu.VMEM((16,), i32),
             pltpu.VMEM((128, 128), f32), pltpu.VMEM((128, 128), f32),
             pltpu.VMEM((128, 128), f32), pltpu.VMEM((128, 128), f32),
             pltpu.VMEM((128, 128), i32), pltpu.VMEM((128, 128), i32),
             pltpu.SemaphoreType.DMA, pltpu.SemaphoreType.DMA,
             pltpu.SemaphoreType.DMA, pltpu.SemaphoreType.DMA,
             pltpu.SemaphoreType.DMA, pltpu.SemaphoreType.DMA,
             pltpu.SemaphoreType.DMA, pltpu.SemaphoreType.DMA,
             pltpu.SemaphoreType.DMA, pltpu.SemaphoreType.DMA,
             pltpu.SemaphoreType.DMA, pltpu.SemaphoreType.DMA,
             pltpu.SemaphoreType.DMA, pltpu.SemaphoreType.DMA,
             pltpu.SemaphoreType.DMA]
  run = pl.kernel(
      _body, out_type=out_type,
      mesh=plsc.VectorSubcoreMesh(core_axis_name="c", subcore_axis_name="s"),
      scratch_types=scratch)

  fs, cr, tr, pe, ok, pp, sc = run(
      pad16(step_indices),
      mem_slot_occupied.reshape(16384, 128), slot_occupied,
      mem_slot_tapped.reshape(16384, 128), slot_tapped,
      mem_game_info.reshape(32768, 128), game_info.reshape(512, 128),
      mos_arg, bos_arg,
      mom_arg, bom_arg,
      mem_lstm_h.reshape(32768, 128), lstm_h_in.reshape(512, 128),
      mem_lstm_c.reshape(32768, 128), lstm_c_in.reshape(512, 128),
      mem_slot_card_rows.reshape(16384, 128), slot_card_rows,
      mok_arg, bok_arg,
      mmay_arg, pad16(may_selected),
      molp_arg, pad16(old_log_probs),
      mval_arg, pad16(values),
      mtr_arg, pad16(trace_kind_id),
      mpe_arg, pad16(pending_kind_id),
      mpp_arg, pad16(perspective_player_idx),
      mem_step_count)
  return (fs.reshape(-1),
          cr.reshape(NUM_ENVS, MAX_STEPS, ZONE_SLOTS),
          tr.reshape(NUM_ENVS, MAX_STEPS),
          pe.reshape(NUM_ENVS, MAX_STEPS),
          ok.reshape(NUM_ENVS, MAX_STEPS, MAX_OPTIONS),
          pp.reshape(NUM_ENVS, MAX_STEPS),
          sc)
